# baseline (speedup 1.0000x reference)
import jax
import jax.numpy as jnp
from jax import lax
from jax.experimental import pallas as pl
from jax.experimental.pallas import tpu as pltpu
from jax.experimental.pallas import tpu_sc as plsc

_B, _D, _K = 4096, 8192, 2048
_L = 16


def _sc_scatter_body(idx_hbm, coef_hbm, w_hbm, idx_v, coef_v, w_v):
    cid = lax.axis_index("c")
    sid = lax.axis_index("s")

    @pl.when(jnp.logical_and(cid == 0, sid == 0))
    def _():
        pltpu.sync_copy(idx_hbm, idx_v)
        pltpu.sync_copy(coef_hbm, coef_v)

        def zero(i, carry):
            for u in range(16):
                w_v[pl.ds((i * 16 + u) * _L, _L)] = jnp.zeros(
                    (_L,), jnp.float32)
            return carry

        lax.fori_loop(0, _D // (16 * _L), zero, 0)

        def acc(i, carry):
            for u in range(16):
                off = (i * 16 + u) * _L
                iv = idx_v[pl.ds(off, _L)]
                cv = coef_v[pl.ds(off, _L)]
                plsc.addupdate_scatter(w_v, [iv], cv)
            return carry

        lax.fori_loop(0, _K // (16 * _L), acc, 0)

        pltpu.sync_copy(w_v, w_hbm)


def _build_w(indices_i32, coef_flat):
    mesh = plsc.VectorSubcoreMesh(core_axis_name="c", subcore_axis_name="s")
    f = pl.kernel(
        _sc_scatter_body,
        out_type=jax.ShapeDtypeStruct((_D,), jnp.float32),
        mesh=mesh,
        compiler_params=pltpu.CompilerParams(needs_layout_passes=False),
        scratch_types=[
            pltpu.VMEM((_K,), jnp.int32),
            pltpu.VMEM((_K,), jnp.float32),
            pltpu.VMEM((_D,), jnp.float32),
        ],
    )
    return f(indices_i32, coef_flat)


_BB = 256


def _tc_mv_body(x_ref, w_ref, icpt_ref, o_ref):
    acc = jnp.sum(x_ref[...] * w_ref[...], axis=1)
    o_ref[...] = acc + icpt_ref[0, 0]


def _tc_matvec(x, w, icpt):
    return pl.pallas_call(
        _tc_mv_body,
        grid=(_B // _BB,),
        in_specs=[
            pl.BlockSpec((_BB, _D), lambda i: (i, 0)),
            pl.BlockSpec((1, _D), lambda i: (0, 0)),
            pl.BlockSpec((1, 1), lambda i: (0, 0)),
        ],
        out_specs=pl.BlockSpec((_BB,), lambda i: (i,)),
        out_shape=jax.ShapeDtypeStruct((_B,), jnp.float32),
    )(x, w.reshape(1, _D), icpt)


def kernel(x, indices, coefficients, intercept):
    idx32 = indices.astype(jnp.int32)
    coef_flat = coefficients.reshape(_K).astype(jnp.float32)
    w = _build_w(idx32, coef_flat)
    icpt = intercept.reshape(1, 1).astype(jnp.float32)
    return _tc_matvec(x, w, icpt)
